# baseline (speedup 1.0000x reference)
import functools

import jax
import jax.numpy as jnp
from jax import lax
from jax.experimental import pallas as pl
from jax.experimental.pallas import tpu as pltpu
from jax.experimental.pallas import tpu_sc as plsc

NW = 10000
NS = 2000
E = 320000
OUT = 64

NTILES = 32
EPT = E // NTILES
C = 80
NCHUNK = EPT // C
SROW = 192
DROW = 208
AROW = 80
NSV = NS // 16


def _prep_words_body(h_ref, w_ref, wgt_ref, s_ref, z_ref):
    z = jnp.dot(h_ref[...], w_ref[...], preferred_element_type=jnp.float32)
    a = jnp.dot(z, wgt_ref[...], preferred_element_type=jnp.float32)
    t = jnp.tanh(z)
    s_ref[...] = jnp.concatenate([z, a, t], axis=1)
    z_ref[...] = z


def _prep_sents_body(o_ref, w1_ref, wgt_ref, wgb_ref, bg_ref, root_ref, d_ref):
    z1 = jnp.dot(o_ref[...], w1_ref[...], preferred_element_type=jnp.float32)
    root = root_ref[...]
    nr = (1.0 - root)[:, None]
    p = (jnp.dot(z1, wgb_ref[...], preferred_element_type=jnp.float32)
         + bg_ref[...][None, :]
         + nr * jnp.dot(z1, wgt_ref[...], preferred_element_type=jnp.float32))
    td = nr * jnp.tanh(z1)
    pad = jnp.zeros((z1.shape[0], 15), jnp.float32)
    d_ref[...] = jnp.concatenate([z1, p, td, root[:, None], pad], axis=1)


def _prep_dfeat_body(t_ref, wf_ref, out_ref):
    out_ref[...] = jnp.dot(t_ref[...], wf_ref[...],
                           preferred_element_type=jnp.float32)


def _combine_body(acc_ref, out_ref):
    a = acc_ref[0] + acc_ref[1]
    num = a[:, :OUT]
    den = jnp.maximum(a[:, OUT], 1e-9)[:, None]
    out_ref[...] = num / den


def _shuf(x, s):
    perm = (lax.iota(jnp.int32, 16) ^ s)[:, None]
    dnums = lax.GatherDimensionNumbers(
        offset_dims=(), collapsed_slice_dims=(0,), start_index_map=(0,))
    return lax.gather(x, perm, dnums, (1,),
                      mode=lax.GatherScatterMode.PROMISE_IN_BOUNDS)


def _lane_sum(x):
    for s in (8, 4, 2, 1):
        x = x + _shuf(x, s)
    return x


def _lane_max(x):
    for s in (8, 4, 2, 1):
        x = jnp.maximum(x, _shuf(x, s))
    return x


def _passa_body(srci_hbm, dsti_hbm, s_hbm, d_hbm, df_hbm, wa_hbm,
                e_hbm, pmax_hbm,
                srci_a, dsti_a, e_a,
                srows0, srows1, drows0, drows1, df0, df1,
                mout_v, wa_v,
                ss0, ss1, sd0, sd1, sf0, sf1):
    wid = lax.axis_index("s") * 2 + lax.axis_index("c")
    tbase = wid * EPT

    pltpu.sync_copy(wa_hbm, wa_v)
    pltpu.sync_copy(srci_hbm.at[pl.ds(tbase, EPT)], srci_a)
    pltpu.sync_copy(dsti_hbm.at[pl.ds(tbase, EPT)], dsti_a)
    wa_c = [wa_v[pl.ds(16 * c, 16)] for c in range(4)]
    lanes = lax.iota(jnp.int32, 16)

    sets = [(srows0, drows0, df0, ss0, sd0, sf0),
            (srows1, drows1, df1, ss1, sd1, sf1)]

    def issue(b, k):
        srows, drows, df, ss, sd, sf = sets[b]
        pltpu.async_copy(s_hbm.at[srci_a.at[pl.ds(k * C, C)]], srows, ss)
        pltpu.async_copy(d_hbm.at[dsti_a.at[pl.ds(k * C, C)]], drows, sd)
        pltpu.async_copy(df_hbm.at[pl.ds((tbase + k * C) // 2, C // 2)],
                         df, sf)

    def wait(b, k):
        srows, drows, df, ss, sd, sf = sets[b]
        pltpu.make_async_copy(
            s_hbm.at[srci_a.at[pl.ds(k * C, C)]], srows, ss).wait()
        pltpu.make_async_copy(
            d_hbm.at[dsti_a.at[pl.ds(k * C, C)]], drows, sd).wait()
        pltpu.make_async_copy(
            df_hbm.at[pl.ds((tbase + k * C) // 2, C // 2)], df, sf).wait()

    def compute(b, k, mmax):
        srows, drows, df, _, _, _ = sets[b]

        def grp(g, mmax_g):
            def edge(jj, carry):
                mcur, eacc = carry
                j = g * 16 + jj
                sv0 = drows[j, pl.ds(192, 16)]
                sv = jnp.full((16,), sv0[0], jnp.float32)
                part = jnp.zeros((16,), jnp.float32)
                for c in range(4):
                    zc = srows[j, pl.ds(16 * c, 16)]
                    ac = srows[j, pl.ds(64 + 16 * c, 16)]
                    tc_ = srows[j, pl.ds(128 + 16 * c, 16)]
                    z1c = drows[j, pl.ds(16 * c, 16)]
                    pc = drows[j, pl.ds(64 + 16 * c, 16)]
                    tdc = drows[j, pl.ds(128 + 16 * c, 16)]
                    dfc = df[j // 2, pl.ds((j % 2) * OUT + 16 * c, 16)]
                    q = jnp.exp(-(sv * ac + pc))
                    tz = sv * tc_ + tdc
                    z22 = (tz + z1c * q) / (1.0 + q)
                    a3 = z22 + zc + dfc
                    y = jnp.maximum(a3, 0.01 * a3)
                    part = part + y * wa_c[c]
                ejv = _lane_sum(part)
                eacc = jnp.where(lanes == jj, ejv, eacc)
                return jnp.maximum(mcur, ejv), eacc
            mmax_g, eacc = lax.fori_loop(
                0, 16, edge, (mmax_g, jnp.zeros((16,), jnp.float32)))
            e_a[pl.ds(k * C + g * 16, 16)] = eacc
            return mmax_g
        return lax.fori_loop(0, C // 16, grp, mmax)

    issue(0, 0)

    def pair(it, mmax):
        ka = 2 * it
        issue(1, ka + 1)
        wait(0, ka)
        mmax = compute(0, ka, mmax)
        issue(0, ka + 2)
        wait(1, ka + 1)
        mmax = compute(1, ka + 1, mmax)
        return mmax
    mmax = lax.fori_loop(0, (NCHUNK - 1) // 2, pair,
                         jnp.full((16,), -jnp.inf, jnp.float32))

    klast = NCHUNK - 1
    wait(0, klast)
    mmax = compute(0, klast, mmax)

    pltpu.sync_copy(e_a, e_hbm.at[pl.ds(tbase, EPT)])
    mout_v[...] = mmax
    pltpu.sync_copy(mout_v, pmax_hbm.at[wid])


def _passb_body(srci_hbm, dsti_hbm, e_hbm, pmax_hbm, z_hbm,
                acc_hbm,
                srci_a, dsti_a, ex_a, pmax_v,
                zrows0, zrows1, row0, row1, dstb0, dstb1,
                zero_v, acc_sh,
                sz0, sz1, sc0, sc1):
    cid = lax.axis_index("c")
    sid = lax.axis_index("s")
    wid = sid * 2 + cid
    tbase = wid * EPT

    pltpu.sync_copy(srci_hbm.at[pl.ds(tbase, EPT)], srci_a)
    pltpu.sync_copy(dsti_hbm.at[pl.ds(tbase, EPT)], dsti_a)
    pltpu.sync_copy(e_hbm.at[pl.ds(tbase, EPT)], ex_a)
    pltpu.sync_copy(pmax_hbm, pmax_v)

    def mrow(r, mcur):
        return jnp.maximum(mcur, pmax_v[r, pl.ds(0, 16)])
    mv = lax.fori_loop(0, NTILES, mrow,
                       jnp.full((16,), -jnp.inf, jnp.float32))
    gmaxv = _lane_max(mv)

    def exv(i, carry):
        sl = pl.ds(i * 16, 16)
        ex_a[sl] = jnp.exp(ex_a[sl] - gmaxv)
        return carry
    lax.fori_loop(0, EPT // 16, exv, 0)

    zeros16 = jnp.zeros((16,), jnp.float32)
    zslice = NS // 16

    def zrow(i, carry):
        zero_v[i // 5, pl.ds((i % 5) * 16, 16)] = zeros16
        return carry
    lax.fori_loop(0, zslice * (AROW // 16), zrow, 0)
    pltpu.sync_copy(zero_v, acc_sh.at[pl.ds(sid * zslice, zslice)])
    plsc.subcore_barrier()

    tailmask = lax.iota(jnp.int32, 16) == 0
    sets = [(zrows0, row0, dstb0, sz0, sc0),
            (zrows1, row1, dstb1, sz1, sc1)]

    def issue(b, k):
        zrows, _, _, sz, _ = sets[b]
        pltpu.async_copy(z_hbm.at[srci_a.at[pl.ds(k * C, C)]], zrows, sz)

    def wait_g(b, k):
        zrows, _, _, sz, _ = sets[b]
        pltpu.make_async_copy(
            z_hbm.at[srci_a.at[pl.ds(k * C, C)]], zrows, sz).wait()

    def wait_sc(b):
        zrows, row, dstb, _, sc = sets[b]
        pltpu.make_async_copy(row, acc_sh.at[dstb], sc).wait()

    def compute(b, k):
        zrows, row, dstb, _, sc = sets[b]

        def grp(g, carry):
            sl16 = pl.ds(k * C + g * 16, 16)
            ex16 = ex_a[sl16]
            dstb[pl.ds(g * 16, 16)] = dsti_a[sl16]
            for jj in range(16):
                j = g * 16 + jj
                exj = jnp.full((16,), ex16[jj], jnp.float32)
                for c in range(4):
                    sl = pl.ds(16 * c, 16)
                    row[j, sl] = exj * zrows[j, sl]
                row[j, pl.ds(64, 16)] = jnp.where(tailmask, exj, 0.0)
            return carry
        lax.fori_loop(0, C // 16, grp, 0)
        pltpu.async_copy(row, acc_sh.at[dstb], sc, add=True)

    issue(0, 0)

    def pair(it, carry):
        ka = 2 * it
        issue(1, ka + 1)
        wait_g(0, ka)

        @pl.when(it > 0)
        def _():
            wait_sc(0)
        compute(0, ka)
        issue(0, ka + 2)
        wait_g(1, ka + 1)

        @pl.when(it > 0)
        def _():
            wait_sc(1)
        compute(1, ka + 1)
        return carry
    lax.fori_loop(0, (NCHUNK - 1) // 2, pair, 0)

    klast = NCHUNK - 1
    wait_g(0, klast)
    wait_sc(0)
    compute(0, klast)
    wait_sc(0)
    wait_sc(1)

    plsc.subcore_barrier()

    @pl.when(sid == 0)
    def _():
        pltpu.sync_copy(acc_sh, acc_hbm.at[cid])


def kernel(h, o, edge_index, tfidfembed, root, W, W1, Wf, Wa, Wg, bg):
    src = edge_index[0]
    dst = edge_index[1]
    wgt = Wg[:OUT]
    wgb = Wg[OUT:]
    wa = Wa[:, 0]

    s_tab, z_tab = pl.pallas_call(
        _prep_words_body,
        grid=(5,),
        in_specs=[
            pl.BlockSpec((2000, 128), lambda i: (i, 0)),
            pl.BlockSpec((128, OUT), lambda i: (0, 0)),
            pl.BlockSpec((OUT, OUT), lambda i: (0, 0)),
        ],
        out_specs=[
            pl.BlockSpec((2000, SROW), lambda i: (i, 0)),
            pl.BlockSpec((2000, OUT), lambda i: (i, 0)),
        ],
        out_shape=[
            jax.ShapeDtypeStruct((NW, SROW), jnp.float32),
            jax.ShapeDtypeStruct((NW, OUT), jnp.float32),
        ],
    )(h, W, wgt)

    d_tab = pl.pallas_call(
        _prep_sents_body,
        out_shape=jax.ShapeDtypeStruct((NS, DROW), jnp.float32),
    )(o, W1, wgt, wgb, bg, root)

    t2 = jnp.reshape(tfidfembed, (E // 2, 32))
    wf2 = jnp.zeros((32, 128), jnp.float32)
    wf2 = wf2.at[:16, :OUT].set(Wf).at[16:, OUT:].set(Wf)
    dfeat = pl.pallas_call(
        _prep_dfeat_body,
        grid=(16,),
        in_specs=[
            pl.BlockSpec((10000, 32), lambda i: (i, 0)),
            pl.BlockSpec((32, 128), lambda i: (0, 0)),
        ],
        out_specs=pl.BlockSpec((10000, 128), lambda i: (i, 0)),
        out_shape=jax.ShapeDtypeStruct((E // 2, 128), jnp.float32),
    )(t2, wf2)

    mesh = plsc.VectorSubcoreMesh(core_axis_name="c", subcore_axis_name="s")
    sc_params = pltpu.CompilerParams(use_tc_tiling_on_sc=False)

    passa = functools.partial(
        pl.kernel,
        out_type=[
            jax.ShapeDtypeStruct((E,), jnp.float32),
            jax.ShapeDtypeStruct((NTILES, 16), jnp.float32),
        ],
        mesh=mesh,
        scratch_types=[
            pltpu.VMEM((EPT,), jnp.int32),
            pltpu.VMEM((EPT,), jnp.int32),
            pltpu.VMEM((EPT,), jnp.float32),
            pltpu.VMEM((C, SROW), jnp.float32),
            pltpu.VMEM((C, SROW), jnp.float32),
            pltpu.VMEM((C, DROW), jnp.float32),
            pltpu.VMEM((C, DROW), jnp.float32),
            pltpu.VMEM((C // 2, 128), jnp.float32),
            pltpu.VMEM((C // 2, 128), jnp.float32),
            pltpu.VMEM((16,), jnp.float32),
            pltpu.VMEM((OUT,), jnp.float32),
            pltpu.SemaphoreType.DMA,
            pltpu.SemaphoreType.DMA,
            pltpu.SemaphoreType.DMA,
            pltpu.SemaphoreType.DMA,
            pltpu.SemaphoreType.DMA,
            pltpu.SemaphoreType.DMA,
        ],
        compiler_params=sc_params,
    )(_passa_body)
    e_arr, pmax = passa(src, dst, s_tab, d_tab, dfeat, wa)

    passb = functools.partial(
        pl.kernel,
        out_type=jax.ShapeDtypeStruct((2, NS, AROW), jnp.float32),
        mesh=mesh,
        scratch_types=[
            pltpu.VMEM((EPT,), jnp.int32),
            pltpu.VMEM((EPT,), jnp.int32),
            pltpu.VMEM((EPT,), jnp.float32),
            pltpu.VMEM((NTILES, 16), jnp.float32),
            pltpu.VMEM((C, OUT), jnp.float32),
            pltpu.VMEM((C, OUT), jnp.float32),
            pltpu.VMEM((C, AROW), jnp.float32),
            pltpu.VMEM((C, AROW), jnp.float32),
            pltpu.VMEM((C,), jnp.int32),
            pltpu.VMEM((C,), jnp.int32),
            pltpu.VMEM((NS // 16, AROW), jnp.float32),
            pltpu.VMEM_SHARED((NS, AROW), jnp.float32),
            pltpu.SemaphoreType.DMA,
            pltpu.SemaphoreType.DMA,
            pltpu.SemaphoreType.DMA,
            pltpu.SemaphoreType.DMA,
        ],
        compiler_params=sc_params,
    )(_passb_body)
    acc = passb(src, dst, e_arr, pmax, z_tab)

    return pl.pallas_call(
        _combine_body,
        out_shape=jax.ShapeDtypeStruct((NS, OUT), jnp.float32),
    )(acc)


# speedup vs baseline: 1.0455x; 1.0455x over previous
import functools

import jax
import jax.numpy as jnp
from jax import lax
from jax.experimental import pallas as pl
from jax.experimental.pallas import tpu as pltpu
from jax.experimental.pallas import tpu_sc as plsc

NW = 10000
NS = 2000
E = 320000
OUT = 64

NTILES = 32
EPT = E // NTILES
C = 80
NCHUNK = EPT // C
SROW = 192
DROW = 208
AROW = 80
NSV = NS // 16


def _prep_words_body(h_ref, w_ref, wgt_ref, s_ref, z_ref):
    z = jnp.dot(h_ref[...], w_ref[...], preferred_element_type=jnp.float32)
    a = jnp.dot(z, wgt_ref[...], preferred_element_type=jnp.float32)
    t = jnp.tanh(z)
    s_ref[...] = jnp.concatenate([z, a, t], axis=1)
    z_ref[...] = z


def _prep_sents_body(o_ref, w1_ref, wgt_ref, wgb_ref, bg_ref, root_ref, d_ref):
    z1 = jnp.dot(o_ref[...], w1_ref[...], preferred_element_type=jnp.float32)
    root = root_ref[...]
    nr = (1.0 - root)[:, None]
    p = (jnp.dot(z1, wgb_ref[...], preferred_element_type=jnp.float32)
         + bg_ref[...][None, :]
         + nr * jnp.dot(z1, wgt_ref[...], preferred_element_type=jnp.float32))
    td = nr * jnp.tanh(z1)
    pad = jnp.zeros((z1.shape[0], 15), jnp.float32)
    d_ref[...] = jnp.concatenate([z1, p, td, root[:, None], pad], axis=1)


def _prep_dfeat_body(tl_ref, tr_ref, wf_ref, out_ref):
    dl = jnp.dot(tl_ref[...], wf_ref[...], preferred_element_type=jnp.float32)
    dr = jnp.dot(tr_ref[...], wf_ref[...], preferred_element_type=jnp.float32)
    out_ref[...] = jnp.concatenate([dl, dr], axis=1)


def _combine_body(acc_ref, out_ref):
    a = acc_ref[0] + acc_ref[1]
    num = a[:, :OUT]
    den = jnp.maximum(a[:, OUT], 1e-9)[:, None]
    out_ref[...] = num / den


def _shuf(x, s):
    perm = (lax.iota(jnp.int32, 16) ^ s)[:, None]
    dnums = lax.GatherDimensionNumbers(
        offset_dims=(), collapsed_slice_dims=(0,), start_index_map=(0,))
    return lax.gather(x, perm, dnums, (1,),
                      mode=lax.GatherScatterMode.PROMISE_IN_BOUNDS)


def _lane_sum(x):
    for s in (8, 4, 2, 1):
        x = x + _shuf(x, s)
    return x


def _lane_max(x):
    for s in (8, 4, 2, 1):
        x = jnp.maximum(x, _shuf(x, s))
    return x


def _passa_body(srci_hbm, dsti_hbm, s_hbm, d_hbm, df_hbm, wa_hbm,
                e_hbm, pmax_hbm,
                srci_a, dsti_a, e_a,
                srows0, srows1, drows0, drows1, df0, df1,
                mout_v, wa_v,
                ss0, ss1, sd0, sd1, sf0, sf1):
    wid = lax.axis_index("s") * 2 + lax.axis_index("c")
    tbase = wid * EPT

    pltpu.sync_copy(wa_hbm, wa_v)
    pltpu.sync_copy(srci_hbm.at[pl.ds(tbase, EPT)], srci_a)
    pltpu.sync_copy(dsti_hbm.at[pl.ds(tbase, EPT)], dsti_a)
    wa_c = [wa_v[pl.ds(16 * c, 16)] for c in range(4)]
    lanes = lax.iota(jnp.int32, 16)

    e2 = E // 2
    second = tbase >= e2
    dfrow = tbase - jnp.where(second, e2, 0)
    dfcol = jnp.where(second, OUT, 0)

    sets = [(srows0, drows0, df0, ss0, sd0, sf0),
            (srows1, drows1, df1, ss1, sd1, sf1)]

    def issue(b, k):
        srows, drows, df, ss, sd, sf = sets[b]
        pltpu.async_copy(s_hbm.at[srci_a.at[pl.ds(k * C, C)]], srows, ss)
        pltpu.async_copy(d_hbm.at[dsti_a.at[pl.ds(k * C, C)]], drows, sd)
        pltpu.async_copy(
            df_hbm.at[pl.ds(dfrow + k * C, C), pl.ds(dfcol, OUT)], df, sf)

    def wait(b, k):
        srows, drows, df, ss, sd, sf = sets[b]
        pltpu.make_async_copy(
            s_hbm.at[srci_a.at[pl.ds(k * C, C)]], srows, ss).wait()
        pltpu.make_async_copy(
            d_hbm.at[dsti_a.at[pl.ds(k * C, C)]], drows, sd).wait()
        pltpu.make_async_copy(
            df_hbm.at[pl.ds(dfrow + k * C, C), pl.ds(dfcol, OUT)],
            df, sf).wait()

    def compute(b, k, mmax):
        srows, drows, df, _, _, _ = sets[b]

        def grp(g, mmax_g):
            def edge(jj, carry):
                mcur, eacc = carry
                j = g * 16 + jj
                sv0 = drows[j, pl.ds(192, 16)]
                sv = jnp.full((16,), sv0[0], jnp.float32)
                part = jnp.zeros((16,), jnp.float32)
                for c in range(4):
                    zc = srows[j, pl.ds(16 * c, 16)]
                    ac = srows[j, pl.ds(64 + 16 * c, 16)]
                    tc_ = srows[j, pl.ds(128 + 16 * c, 16)]
                    z1c = drows[j, pl.ds(16 * c, 16)]
                    pc = drows[j, pl.ds(64 + 16 * c, 16)]
                    tdc = drows[j, pl.ds(128 + 16 * c, 16)]
                    dfc = df[j, pl.ds(16 * c, 16)]
                    q = jnp.exp(-(sv * ac + pc))
                    tz = sv * tc_ + tdc
                    z22 = (tz + z1c * q) / (1.0 + q)
                    a3 = z22 + zc + dfc
                    y = jnp.maximum(a3, 0.01 * a3)
                    part = part + y * wa_c[c]
                ejv = _lane_sum(part)
                eacc = jnp.where(lanes == jj, ejv, eacc)
                return jnp.maximum(mcur, ejv), eacc
            mmax_g, eacc = lax.fori_loop(
                0, 16, edge, (mmax_g, jnp.zeros((16,), jnp.float32)))
            e_a[pl.ds(k * C + g * 16, 16)] = eacc
            return mmax_g
        return lax.fori_loop(0, C // 16, grp, mmax)

    issue(0, 0)

    def pair(it, mmax):
        ka = 2 * it
        issue(1, ka + 1)
        wait(0, ka)
        mmax = compute(0, ka, mmax)
        issue(0, ka + 2)
        wait(1, ka + 1)
        mmax = compute(1, ka + 1, mmax)
        return mmax
    mmax = lax.fori_loop(0, (NCHUNK - 1) // 2, pair,
                         jnp.full((16,), -jnp.inf, jnp.float32))

    klast = NCHUNK - 1
    wait(0, klast)
    mmax = compute(0, klast, mmax)

    pltpu.sync_copy(e_a, e_hbm.at[pl.ds(tbase, EPT)])
    mout_v[...] = mmax
    pltpu.sync_copy(mout_v, pmax_hbm.at[wid])


def _passb_body(srci_hbm, dsti_hbm, e_hbm, pmax_hbm, z_hbm,
                acc_hbm,
                srci_a, dsti_a, ex_a, pmax_v,
                zrows0, zrows1, row0, row1, dstb0, dstb1,
                zero_v, acc_sh,
                sz0, sz1, sc0, sc1):
    cid = lax.axis_index("c")
    sid = lax.axis_index("s")
    wid = sid * 2 + cid
    tbase = wid * EPT

    pltpu.sync_copy(srci_hbm.at[pl.ds(tbase, EPT)], srci_a)
    pltpu.sync_copy(dsti_hbm.at[pl.ds(tbase, EPT)], dsti_a)
    pltpu.sync_copy(e_hbm.at[pl.ds(tbase, EPT)], ex_a)
    pltpu.sync_copy(pmax_hbm, pmax_v)

    def mrow(r, mcur):
        return jnp.maximum(mcur, pmax_v[r, pl.ds(0, 16)])
    mv = lax.fori_loop(0, NTILES, mrow,
                       jnp.full((16,), -jnp.inf, jnp.float32))
    gmaxv = _lane_max(mv)

    def exv(i, carry):
        sl = pl.ds(i * 16, 16)
        ex_a[sl] = jnp.exp(ex_a[sl] - gmaxv)
        return carry
    lax.fori_loop(0, EPT // 16, exv, 0)

    zeros16 = jnp.zeros((16,), jnp.float32)
    zslice = NS // 16

    def zrow(i, carry):
        zero_v[i // 5, pl.ds((i % 5) * 16, 16)] = zeros16
        return carry
    lax.fori_loop(0, zslice * (AROW // 16), zrow, 0)
    pltpu.sync_copy(zero_v, acc_sh.at[pl.ds(sid * zslice, zslice)])
    plsc.subcore_barrier()

    tailmask = lax.iota(jnp.int32, 16) == 0
    sets = [(zrows0, row0, dstb0, sz0, sc0),
            (zrows1, row1, dstb1, sz1, sc1)]

    def issue(b, k):
        zrows, _, _, sz, _ = sets[b]
        pltpu.async_copy(z_hbm.at[srci_a.at[pl.ds(k * C, C)]], zrows, sz)

    def wait_g(b, k):
        zrows, _, _, sz, _ = sets[b]
        pltpu.make_async_copy(
            z_hbm.at[srci_a.at[pl.ds(k * C, C)]], zrows, sz).wait()

    def wait_sc(b):
        zrows, row, dstb, _, sc = sets[b]
        pltpu.make_async_copy(row, acc_sh.at[dstb], sc).wait()

    def compute(b, k):
        zrows, row, dstb, _, sc = sets[b]

        def grp(g, carry):
            sl16 = pl.ds(k * C + g * 16, 16)
            ex16 = ex_a[sl16]
            dstb[pl.ds(g * 16, 16)] = dsti_a[sl16]
            for jj in range(16):
                j = g * 16 + jj
                exj = jnp.full((16,), ex16[jj], jnp.float32)
                for c in range(4):
                    sl = pl.ds(16 * c, 16)
                    row[j, sl] = exj * zrows[j, sl]
                row[j, pl.ds(64, 16)] = jnp.where(tailmask, exj, 0.0)
            return carry
        lax.fori_loop(0, C // 16, grp, 0)
        pltpu.async_copy(row, acc_sh.at[dstb], sc, add=True)

    issue(0, 0)

    def pair(it, carry):
        ka = 2 * it
        issue(1, ka + 1)
        wait_g(0, ka)

        @pl.when(it > 0)
        def _():
            wait_sc(0)
        compute(0, ka)
        issue(0, ka + 2)
        wait_g(1, ka + 1)

        @pl.when(it > 0)
        def _():
            wait_sc(1)
        compute(1, ka + 1)
        return carry
    lax.fori_loop(0, (NCHUNK - 1) // 2, pair, 0)

    klast = NCHUNK - 1
    wait_g(0, klast)
    wait_sc(0)
    compute(0, klast)
    wait_sc(0)
    wait_sc(1)

    plsc.subcore_barrier()

    @pl.when(sid == 0)
    def _():
        pltpu.sync_copy(acc_sh, acc_hbm.at[cid])


def kernel(h, o, edge_index, tfidfembed, root, W, W1, Wf, Wa, Wg, bg):
    src = edge_index[0]
    dst = edge_index[1]
    wgt = Wg[:OUT]
    wgb = Wg[OUT:]
    wa = Wa[:, 0]

    s_tab, z_tab = pl.pallas_call(
        _prep_words_body,
        grid=(5,),
        in_specs=[
            pl.BlockSpec((2000, 128), lambda i: (i, 0)),
            pl.BlockSpec((128, OUT), lambda i: (0, 0)),
            pl.BlockSpec((OUT, OUT), lambda i: (0, 0)),
        ],
        out_specs=[
            pl.BlockSpec((2000, SROW), lambda i: (i, 0)),
            pl.BlockSpec((2000, OUT), lambda i: (i, 0)),
        ],
        out_shape=[
            jax.ShapeDtypeStruct((NW, SROW), jnp.float32),
            jax.ShapeDtypeStruct((NW, OUT), jnp.float32),
        ],
    )(h, W, wgt)

    d_tab = pl.pallas_call(
        _prep_sents_body,
        out_shape=jax.ShapeDtypeStruct((NS, DROW), jnp.float32),
    )(o, W1, wgt, wgb, bg, root)

    dfeat = pl.pallas_call(
        _prep_dfeat_body,
        grid=(16,),
        in_specs=[
            pl.BlockSpec((10000, 16), lambda i: (i, 0)),
            pl.BlockSpec((10000, 16), lambda i: (i + 16, 0)),
            pl.BlockSpec((16, OUT), lambda i: (0, 0)),
        ],
        out_specs=pl.BlockSpec((10000, 128), lambda i: (i, 0)),
        out_shape=jax.ShapeDtypeStruct((E // 2, 128), jnp.float32),
    )(tfidfembed, tfidfembed, Wf)

    mesh = plsc.VectorSubcoreMesh(core_axis_name="c", subcore_axis_name="s")
    sc_params = pltpu.CompilerParams(use_tc_tiling_on_sc=False)

    passa = functools.partial(
        pl.kernel,
        out_type=[
            jax.ShapeDtypeStruct((E,), jnp.float32),
            jax.ShapeDtypeStruct((NTILES, 16), jnp.float32),
        ],
        mesh=mesh,
        scratch_types=[
            pltpu.VMEM((EPT,), jnp.int32),
            pltpu.VMEM((EPT,), jnp.int32),
            pltpu.VMEM((EPT,), jnp.float32),
            pltpu.VMEM((C, SROW), jnp.float32),
            pltpu.VMEM((C, SROW), jnp.float32),
            pltpu.VMEM((C, DROW), jnp.float32),
            pltpu.VMEM((C, DROW), jnp.float32),
            pltpu.VMEM((C, OUT), jnp.float32),
            pltpu.VMEM((C, OUT), jnp.float32),
            pltpu.VMEM((16,), jnp.float32),
            pltpu.VMEM((OUT,), jnp.float32),
            pltpu.SemaphoreType.DMA,
            pltpu.SemaphoreType.DMA,
            pltpu.SemaphoreType.DMA,
            pltpu.SemaphoreType.DMA,
            pltpu.SemaphoreType.DMA,
            pltpu.SemaphoreType.DMA,
        ],
        compiler_params=sc_params,
    )(_passa_body)
    e_arr, pmax = passa(src, dst, s_tab, d_tab, dfeat, wa)

    passb = functools.partial(
        pl.kernel,
        out_type=jax.ShapeDtypeStruct((2, NS, AROW), jnp.float32),
        mesh=mesh,
        scratch_types=[
            pltpu.VMEM((EPT,), jnp.int32),
            pltpu.VMEM((EPT,), jnp.int32),
            pltpu.VMEM((EPT,), jnp.float32),
            pltpu.VMEM((NTILES, 16), jnp.float32),
            pltpu.VMEM((C, OUT), jnp.float32),
            pltpu.VMEM((C, OUT), jnp.float32),
            pltpu.VMEM((C, AROW), jnp.float32),
            pltpu.VMEM((C, AROW), jnp.float32),
            pltpu.VMEM((C,), jnp.int32),
            pltpu.VMEM((C,), jnp.int32),
            pltpu.VMEM((NS // 16, AROW), jnp.float32),
            pltpu.VMEM_SHARED((NS, AROW), jnp.float32),
            pltpu.SemaphoreType.DMA,
            pltpu.SemaphoreType.DMA,
            pltpu.SemaphoreType.DMA,
            pltpu.SemaphoreType.DMA,
        ],
        compiler_params=sc_params,
    )(_passb_body)
    acc = passb(src, dst, e_arr, pmax, z_tab)

    return pl.pallas_call(
        _combine_body,
        out_shape=jax.ShapeDtypeStruct((NS, OUT), jnp.float32),
    )(acc)
